# baseline (speedup 1.0000x reference)
import functools

import jax
import jax.numpy as jnp
from jax import lax
from jax.experimental import pallas as pl
from jax.experimental.pallas import tpu as pltpu
from jax.experimental.pallas import tpu_sc as plsc

_B, _S, _D = 4, 4096, 2048
_NC, _NS = 2, 16
_NW = _NC * _NS
_ROWS = _B * _S
_RPW = _ROWS // _NW
_CHUNK = 16
_NBUF = 3
_NCHUNK = _RPW // _CHUNK
_LANES = 16


def _gather_body(x_hbm, perm_hbm, out_hbm, idx_v, buf_v, *sems):
    sem_g = sems[:_NBUF]
    sem_s = sems[_NBUF:]
    cid = lax.axis_index("c")
    sid = lax.axis_index("s")
    wid = sid * _NC + cid
    base = wid * _RPW
    b = base // _S
    i0 = base - b * _S
    off = b * _S

    pltpu.sync_copy(perm_hbm.at[pl.ds(i0, _RPW)], idx_v)
    off_vec = jnp.full((_LANES,), off, dtype=jnp.int32)
    for j in range(_RPW // _LANES):
        sl = pl.ds(j * _LANES, _LANES)
        idx_v[sl] = idx_v[sl] + off_vec

    def g_copy(g, slot):
        idx_slice = idx_v.at[pl.ds(g * _CHUNK, _CHUNK)]
        return pltpu.make_async_copy(
            x_hbm.at[idx_slice], buf_v.at[slot], sem_g[slot])

    def s_copy(g, slot):
        return pltpu.make_async_copy(
            buf_v.at[slot], out_hbm.at[pl.ds(base + g * _CHUNK, _CHUNK)],
            sem_s[slot])

    def position(g):
        pf = g + 1
        if isinstance(g, int):
            if pf < _NCHUNK:
                if pf - _NBUF >= 0:
                    s_copy(pf - _NBUF, pf % _NBUF).wait()
                g_copy(pf, pf % _NBUF).start()
            g_copy(g, g % _NBUF).wait()
            s_copy(g, g % _NBUF).start()

    g_copy(0, 0).start()
    position(0)
    position(1)
    position(2)

    def steady(t, carry):
        for b in range(_NBUF):
            g = 3 + t * _NBUF + b
            s_copy(g + 1 - _NBUF, (b + 1) % _NBUF).wait()
            g_copy(g + 1, (b + 1) % _NBUF).start()
            g_copy(g, b).wait()
            s_copy(g, b).start()
        return carry

    lax.fori_loop(0, (_NCHUNK - 5) // _NBUF, steady, 0)

    position(_NCHUNK - 2)
    position(_NCHUNK - 1)
    for g in range(_NCHUNK - _NBUF, _NCHUNK):
        s_copy(g, g % _NBUF).wait()


@jax.jit
def kernel(x, perm):
    x2 = x.reshape(_ROWS, _D)
    p32 = perm.astype(jnp.int32)
    mesh = plsc.VectorSubcoreMesh(core_axis_name="c", subcore_axis_name="s")
    run = pl.kernel(
        _gather_body,
        mesh=mesh,
        out_type=jax.ShapeDtypeStruct((_ROWS, _D), jnp.float32),
        scratch_types=[
            pltpu.VMEM((_RPW,), jnp.int32),
            pltpu.VMEM((_NBUF, _CHUNK, _D), jnp.float32),
        ] + [pltpu.SemaphoreType.DMA] * (2 * _NBUF),
    )
    out = run(x2, p32)
    return out.reshape(_B, _S, _D)
